# Initial kernel scaffold; baseline (speedup 1.0000x reference)
import jax
import jax.numpy as jnp
from jax.experimental import pallas as pl


def kernel(hidden_states, weight):
    raise NotImplementedError("write your pallas kernel here")


# speedup vs baseline: 1.0577x; 1.0577x over previous
import functools

import jax
import jax.numpy as jnp
from jax.experimental import pallas as pl

_TOP_K = 8
_NEG_INF = float("-inf")


def _gate_body(x_ref, w_ref, idx_ref, wgt_ref):
    x = x_ref[:]
    w = w_ref[:]
    logits = jax.lax.dot_general(
        x, w, (((1,), (1,)), ((), ())), preferred_element_type=jnp.float32
    )

    bt, e = logits.shape
    lane = jax.lax.broadcasted_iota(jnp.int32, (bt, e), 1)
    cur = logits
    vals = []
    idxs = []
    for _ in range(_TOP_K):
        m = jnp.max(cur, axis=-1, keepdims=True)
        is_max = cur == m
        a = jnp.min(jnp.where(is_max, lane, e), axis=-1, keepdims=True)
        vals.append(m)
        idxs.append(a)
        cur = jnp.where(lane == a, _NEG_INF, cur)

    topv = jnp.concatenate(vals, axis=-1)
    topi = jnp.concatenate(idxs, axis=-1)

    ex = jnp.exp(topv - topv[:, 0:1])
    wgt = ex / jnp.sum(ex, axis=-1, keepdims=True)

    idx_ref[:] = topi
    wgt_ref[:] = wgt


@functools.partial(jax.jit, static_argnames=())
def _gate(flat, weight):
    t, h = flat.shape
    e = weight.shape[0]
    bt = 512
    grid = (t // bt,)
    topi, topw = pl.pallas_call(
        _gate_body,
        grid=grid,
        in_specs=[
            pl.BlockSpec((bt, h), lambda i: (i, 0)),
            pl.BlockSpec((e, h), lambda i: (0, 0)),
        ],
        out_specs=[
            pl.BlockSpec((bt, _TOP_K), lambda i: (i, 0)),
            pl.BlockSpec((bt, _TOP_K), lambda i: (i, 0)),
        ],
        out_shape=[
            jax.ShapeDtypeStruct((t, _TOP_K), jnp.int32),
            jax.ShapeDtypeStruct((t, _TOP_K), jnp.float32),
        ],
        compiler_params=pltpu_params(),
    )(flat, weight)
    return topi, topw


def pltpu_params():
    from jax.experimental.pallas import tpu as pltpu

    return pltpu.CompilerParams(dimension_semantics=("arbitrary",))


def kernel(hidden_states, weight):
    bsz, seq_len, h = hidden_states.shape
    flat = hidden_states.reshape(-1, h)
    topi, topw = _gate(flat, weight)
    aux_loss = jnp.float32(0.0)
    return (topi, topw, aux_loss)


# speedup vs baseline: 1.2281x; 1.1612x over previous
import functools

import jax
import jax.numpy as jnp
from jax.experimental import pallas as pl

_TOP_K = 8
_NEG_INF = float("-inf")


def _gate_body(x_ref, w_ref, idx_ref, wgt_ref):
    x = x_ref[:]
    w = w_ref[:]
    logits = jax.lax.dot_general(
        x, w, (((1,), (1,)), ((), ())), preferred_element_type=jnp.float32
    )

    bt, e = logits.shape
    lane = jax.lax.broadcasted_iota(jnp.int32, (bt, e), 1)

    bits = jax.lax.bitcast_convert_type(logits, jnp.int32)
    sign = jax.lax.shift_right_arithmetic(bits, 31)
    skey = jnp.bitwise_xor(bits, jnp.bitwise_and(sign, jnp.int32(0x7FFFFFFF)))
    key = jnp.bitwise_or(
        jnp.bitwise_and(skey, jnp.int32(~0x3F)), jnp.int32(e - 1) - lane
    )

    keys = []
    cur = key
    for _ in range(_TOP_K):
        m = jnp.max(cur, axis=-1, keepdims=True)
        keys.append(m)
        cur = jnp.where(cur == m, jnp.int32(-0x80000000), cur)

    topk = jnp.concatenate(keys, axis=-1)
    topi = jnp.int32(e - 1) - jnp.bitwise_and(topk, jnp.int32(0x3F))

    vkey = jnp.bitwise_and(topk, jnp.int32(~0x3F))
    vsign = jax.lax.shift_right_arithmetic(vkey, 31)
    vbits = jnp.bitwise_xor(vkey, jnp.bitwise_and(vsign, jnp.int32(0x7FFFFFFF)))
    topv = jax.lax.bitcast_convert_type(vbits, jnp.float32)

    ex = jnp.exp(topv - topv[:, 0:1])
    wgt = ex / jnp.sum(ex, axis=-1, keepdims=True)

    idx_ref[:] = topi
    wgt_ref[:] = wgt


@functools.partial(jax.jit, static_argnames=())
def _gate(flat, weight):
    t, h = flat.shape
    e = weight.shape[0]
    bt = 512
    grid = (t // bt,)
    topi, topw = pl.pallas_call(
        _gate_body,
        grid=grid,
        in_specs=[
            pl.BlockSpec((bt, h), lambda i: (i, 0)),
            pl.BlockSpec((e, h), lambda i: (0, 0)),
        ],
        out_specs=[
            pl.BlockSpec((bt, _TOP_K), lambda i: (i, 0)),
            pl.BlockSpec((bt, _TOP_K), lambda i: (i, 0)),
        ],
        out_shape=[
            jax.ShapeDtypeStruct((t, _TOP_K), jnp.int32),
            jax.ShapeDtypeStruct((t, _TOP_K), jnp.float32),
        ],
        compiler_params=pltpu_params(),
    )(flat, weight)
    return topi, topw


def pltpu_params():
    from jax.experimental.pallas import tpu as pltpu

    return pltpu.CompilerParams(dimension_semantics=("arbitrary",))


def kernel(hidden_states, weight):
    bsz, seq_len, h = hidden_states.shape
    flat = hidden_states.reshape(-1, h)
    topi, topw = _gate(flat, weight)
    aux_loss = jnp.float32(0.0)
    return (topi, topw, aux_loss)


# speedup vs baseline: 1.3421x; 1.0928x over previous
import functools

import jax
import jax.numpy as jnp
from jax.experimental import pallas as pl

_TOP_K = 8
_NEG_INF = float("-inf")


def _gate_body(x_ref, w_ref, idx_ref, wgt_ref):
    x = x_ref[:]
    w = w_ref[:]
    logits = jax.lax.dot_general(
        x, w, (((1,), (1,)), ((), ())), preferred_element_type=jnp.float32
    )

    bt, e = logits.shape
    lane = jax.lax.broadcasted_iota(jnp.int32, (bt, e), 1)

    bits = jax.lax.bitcast_convert_type(logits, jnp.int32)
    sign = jax.lax.shift_right_arithmetic(bits, 31)
    tag = jnp.bitwise_xor(jnp.int32(e - 1) - lane, jnp.bitwise_and(sign, 0x3F))
    kbits = jnp.bitwise_or(jnp.bitwise_and(bits, jnp.int32(~0x3F)), tag)
    key = jax.lax.bitcast_convert_type(kbits, jnp.float32)

    keys = []
    cur = key
    for _ in range(_TOP_K):
        m = jnp.max(cur, axis=-1, keepdims=True)
        keys.append(m)
        cur = jnp.where(cur == m, _NEG_INF, cur)

    topk = jnp.concatenate(keys, axis=-1)
    tbits = jax.lax.bitcast_convert_type(topk, jnp.int32)
    tsign = jax.lax.shift_right_arithmetic(tbits, 31)
    ttag = jnp.bitwise_xor(
        jnp.bitwise_and(tbits, jnp.int32(0x3F)), jnp.bitwise_and(tsign, 0x3F)
    )
    topi = jnp.int32(e - 1) - ttag

    topv = jax.lax.bitcast_convert_type(
        jnp.bitwise_and(tbits, jnp.int32(~0x3F)), jnp.float32
    )

    ex = jnp.exp(topv - topv[:, 0:1])
    wgt = ex / jnp.sum(ex, axis=-1, keepdims=True)

    idx_ref[:] = topi
    wgt_ref[:] = wgt


@functools.partial(jax.jit, static_argnames=())
def _gate(flat, weight):
    t, h = flat.shape
    e = weight.shape[0]
    bt = 512
    grid = (t // bt,)
    topi, topw = pl.pallas_call(
        _gate_body,
        grid=grid,
        in_specs=[
            pl.BlockSpec((bt, h), lambda i: (i, 0)),
            pl.BlockSpec((e, h), lambda i: (0, 0)),
        ],
        out_specs=[
            pl.BlockSpec((bt, _TOP_K), lambda i: (i, 0)),
            pl.BlockSpec((bt, _TOP_K), lambda i: (i, 0)),
        ],
        out_shape=[
            jax.ShapeDtypeStruct((t, _TOP_K), jnp.int32),
            jax.ShapeDtypeStruct((t, _TOP_K), jnp.float32),
        ],
        compiler_params=pltpu_params(),
    )(flat, weight)
    return topi, topw


def pltpu_params():
    from jax.experimental.pallas import tpu as pltpu

    return pltpu.CompilerParams(dimension_semantics=("arbitrary",))


def kernel(hidden_states, weight):
    bsz, seq_len, h = hidden_states.shape
    flat = hidden_states.reshape(-1, h)
    topi, topw = _gate(flat, weight)
    aux_loss = jnp.float32(0.0)
    return (topi, topw, aux_loss)


# speedup vs baseline: 1.4757x; 1.0995x over previous
import functools

import jax
import jax.numpy as jnp
from jax.experimental import pallas as pl

_TOP_K = 8
_NEG_INF = float("-inf")


def _gate_body(x_ref, w_ref, idx_ref, wgt_ref):
    x = x_ref[:]
    w = w_ref[:]
    logits = jax.lax.dot_general(
        x, w, (((1,), (1,)), ((), ())), preferred_element_type=jnp.float32
    )

    bt, e = logits.shape
    lane = jax.lax.broadcasted_iota(jnp.int32, (bt, e), 1)

    bits = jax.lax.bitcast_convert_type(logits, jnp.int32)
    sign = jax.lax.shift_right_arithmetic(bits, 31)
    tag = jnp.bitwise_xor(jnp.int32(e - 1) - lane, jnp.bitwise_and(sign, 0x3F))
    kbits = jnp.bitwise_or(jnp.bitwise_and(bits, jnp.int32(~0x3F)), tag)
    key = jax.lax.bitcast_convert_type(kbits, jnp.float32)

    keys = []
    cur = key
    for _ in range(_TOP_K):
        m = jnp.max(cur, axis=-1, keepdims=True)
        keys.append(m)
        cur = jnp.where(cur == m, _NEG_INF, cur)

    topk = jnp.concatenate(keys, axis=-1)
    tbits = jax.lax.bitcast_convert_type(topk, jnp.int32)
    tsign = jax.lax.shift_right_arithmetic(tbits, 31)
    ttag = jnp.bitwise_xor(
        jnp.bitwise_and(tbits, jnp.int32(0x3F)), jnp.bitwise_and(tsign, 0x3F)
    )
    topi = jnp.int32(e - 1) - ttag

    topv = jax.lax.bitcast_convert_type(
        jnp.bitwise_and(tbits, jnp.int32(~0x3F)), jnp.float32
    )

    ex = jnp.exp(topv - topv[:, 0:1])
    wgt = ex / jnp.sum(ex, axis=-1, keepdims=True)

    idx_ref[:] = topi
    wgt_ref[:] = wgt


@functools.partial(jax.jit, static_argnames=())
def _gate(flat, weight):
    t, h = flat.shape
    e = weight.shape[0]
    bt = 1024
    grid = (t // bt,)
    topi, topw = pl.pallas_call(
        _gate_body,
        grid=grid,
        in_specs=[
            pl.BlockSpec((bt, h), lambda i: (i, 0)),
            pl.BlockSpec((e, h), lambda i: (0, 0)),
        ],
        out_specs=[
            pl.BlockSpec((bt, _TOP_K), lambda i: (i, 0)),
            pl.BlockSpec((bt, _TOP_K), lambda i: (i, 0)),
        ],
        out_shape=[
            jax.ShapeDtypeStruct((t, _TOP_K), jnp.int32),
            jax.ShapeDtypeStruct((t, _TOP_K), jnp.float32),
        ],
        compiler_params=pltpu_params(),
    )(flat, weight)
    return topi, topw


def pltpu_params():
    from jax.experimental.pallas import tpu as pltpu

    return pltpu.CompilerParams(dimension_semantics=("arbitrary",))


def kernel(hidden_states, weight):
    bsz, seq_len, h = hidden_states.shape
    flat = hidden_states.reshape(-1, h)
    topi, topw = _gate(flat, weight)
    aux_loss = jnp.float32(0.0)
    return (topi, topw, aux_loss)
